# baseline (speedup 1.0000x reference)
import jax
import jax.numpy as jnp
from jax import lax
from jax.experimental import pallas as pl
from jax.experimental.pallas import tpu as pltpu
from jax.experimental.pallas import tpu_sc as plsc

D_MODEL = 64
BATCH = 16384
HIST = 50
HIST_PAD = 56
D_PAD = 128
B_TOTAL = BATCH * HIST
NUM_CORES = 2
NUM_SUBCORES = 16
NW = NUM_CORES * NUM_SUBCORES
B_PER_W = B_TOTAL // NW
XROW_PER_W = BATCH // NW
XROW_CHUNK = 8
CHUNK = XROW_CHUNK * HIST
N_CHUNK = XROW_PER_W // XROW_CHUNK
NBUF = 3


def _emb_body(table_hbm, idx_hbm, out_hbm, idx_v, rows_v, sem_g, sem_o):
    wid = lax.axis_index("s") * NUM_CORES + lax.axis_index("c")
    base = wid * B_PER_W
    xbase = wid * XROW_PER_W

    pltpu.sync_copy(idx_hbm.at[pl.ds(base, B_PER_W)], idx_v)

    def idx_slice(j):
        return idx_v.at[pl.ds(j * CHUNK, CHUNK)]

    def start_gather(j, b):
        pltpu.async_copy(table_hbm.at[idx_slice(j)], rows_v[b], sem_g[b])

    def wait_gather(j, b):
        pltpu.make_async_copy(
            table_hbm.at[idx_slice(j)], rows_v[b], sem_g[b]).wait()

    def out_pairs(j, b):
        for i in range(XROW_CHUNK):
            row0 = (xbase + j * XROW_CHUNK + i) * HIST_PAD
            src = rows_v[b].at[pl.ds(i * HIST, HIST), :]
            dst = out_hbm.at[pl.ds(row0, HIST), pl.ds(0, D_MODEL)]
            yield src, dst

    def start_out(j, b):
        for src, dst in out_pairs(j, b):
            pltpu.async_copy(src, dst, sem_o[b])

    def wait_out(j, b):
        for src, dst in out_pairs(j, b):
            pltpu.make_async_copy(src, dst, sem_o[b]).wait()

    for b in range(NBUF - 1):
        start_gather(b, b)

    def step(g, carry):
        for b in range(NBUF):
            j = g * NBUF + b

            @pl.when(j < N_CHUNK)
            def _():
                wait_gather(j, b)
                start_out(j, b)
                bp = (b - 1) % NBUF
                jn = j + NBUF - 1

                @pl.when(jn < N_CHUNK)
                def _():
                    @pl.when(j >= 1)
                    def _():
                        wait_out(j - 1, bp)

                    start_gather(jn, bp)

        return carry

    lax.fori_loop(0, pl.cdiv(N_CHUNK, NBUF), step, 0)

    for b in range(NBUF):
        j_last = N_CHUNK - NBUF + b
        wait_out(j_last, j_last % NBUF)


def kernel(x, weight):
    idx2 = x.reshape(-1)
    wview = weight
    mesh = plsc.VectorSubcoreMesh(core_axis_name="c", subcore_axis_name="s")
    out = pl.kernel(
        _emb_body,
        mesh=mesh,
        out_type=jax.ShapeDtypeStruct((BATCH * HIST_PAD, D_PAD), jnp.float32),
        scratch_types=[
            pltpu.VMEM((B_PER_W,), jnp.int32),
            [pltpu.VMEM((CHUNK, D_MODEL), jnp.float32) for _ in range(NBUF)],
            [pltpu.SemaphoreType.DMA for _ in range(NBUF)],
            [pltpu.SemaphoreType.DMA for _ in range(NBUF)],
        ],
        compiler_params=pltpu.CompilerParams(use_tc_tiling_on_sc=False),
    )(wview, idx2)
    return out.reshape(BATCH, HIST_PAD, D_PAD)[:, :HIST, :D_MODEL]


# speedup vs baseline: 1.1183x; 1.1183x over previous
import jax
import jax.numpy as jnp
from jax import lax
from jax.experimental import pallas as pl
from jax.experimental.pallas import tpu as pltpu
from jax.experimental.pallas import tpu_sc as plsc

D_MODEL = 64
BATCH = 16384
HIST = 50
HIST_PAD = 56
D_PAD = 128
B_TOTAL = BATCH * HIST
NUM_CORES = 2
NUM_SUBCORES = 16
NW = NUM_CORES * NUM_SUBCORES
B_PER_W = B_TOTAL // NW
XROW_PER_W = BATCH // NW
XROW_CHUNK = 8
CHUNK = XROW_CHUNK * HIST
N_CHUNK = XROW_PER_W // XROW_CHUNK
NBUF = 3


def _emb_body(table_hbm, idx_hbm, out_hbm, idx_v, rows_v, sem_g, sem_o):
    wid = lax.axis_index("s") * NUM_CORES + lax.axis_index("c")
    base = wid * B_PER_W
    xbase = wid * XROW_PER_W

    pltpu.sync_copy(idx_hbm.at[pl.ds(base, B_PER_W)], idx_v)

    def idx_slice(j):
        return idx_v.at[pl.ds(j * CHUNK, CHUNK)]

    def start_gather(j, b):
        pltpu.async_copy(table_hbm.at[idx_slice(j)], rows_v[b], sem_g[b])

    def wait_gather(j, b):
        pltpu.make_async_copy(
            table_hbm.at[idx_slice(j)], rows_v[b], sem_g[b]).wait()

    def out_pairs(j, b):
        for i in range(XROW_CHUNK):
            row0 = (xbase + j * XROW_CHUNK + i) * HIST_PAD
            src = rows_v[b].at[pl.ds(i * HIST, HIST), :]
            dst = out_hbm.at[pl.ds(row0, HIST), pl.ds(0, D_MODEL)]
            yield src, dst

    def start_out(j, b):
        for src, dst in out_pairs(j, b):
            pltpu.async_copy(src, dst, sem_o[b])

    def wait_out(j, b):
        for src, dst in out_pairs(j, b):
            pltpu.make_async_copy(src, dst, sem_o[b]).wait()

    for b in range(NBUF - 1):
        start_gather(b, b)

    def step(g, carry):
        for b in range(NBUF):
            j = g * NBUF + b

            @pl.when(j < N_CHUNK)
            def _():
                wait_gather(j, b)
                start_out(j, b)
                bp = (b - 1) % NBUF
                jn = j + NBUF - 1

                @pl.when(jn < N_CHUNK)
                def _():
                    @pl.when(j >= 1)
                    def _():
                        wait_out(j - 1, bp)

                    start_gather(jn, bp)

        return carry

    lax.fori_loop(0, pl.cdiv(N_CHUNK, NBUF), step, 0)

    for b in range(NBUF):
        j_last = N_CHUNK - NBUF + b
        wait_out(j_last, j_last % NBUF)


TR_BK = 2048


def _tr_body(wt_ref, out_ref):
    eye = jax.lax.broadcasted_iota(jnp.int32, (D_MODEL, D_MODEL), 0)
    eyef = jnp.where(eye == jax.lax.broadcasted_iota(
        jnp.int32, (D_MODEL, D_MODEL), 1), 1.0, 0.0).astype(jnp.float32)
    out_ref[:, :D_MODEL] = jax.lax.dot_general(
        wt_ref[...], eyef,
        dimension_numbers=(((0,), (0,)), ((), ())),
        preferred_element_type=jnp.float32)


def _pad_transpose(wt):
    grid = pl.cdiv(1000000, TR_BK)
    return pl.pallas_call(
        _tr_body,
        grid=(grid,),
        in_specs=[pl.BlockSpec((D_MODEL, TR_BK), lambda i: (0, i))],
        out_specs=pl.BlockSpec((TR_BK, D_PAD), lambda i: (i, 0)),
        out_shape=jax.ShapeDtypeStruct((1000000, D_PAD), jnp.float32),
    )(wt)


def kernel(x, weight):
    idx2 = x.reshape(-1) * 2
    wview = _pad_transpose(weight.T).reshape(2 * 1000000, D_MODEL)
    mesh = plsc.VectorSubcoreMesh(core_axis_name="c", subcore_axis_name="s")
    out = pl.kernel(
        _emb_body,
        mesh=mesh,
        out_type=jax.ShapeDtypeStruct((BATCH * HIST_PAD, D_PAD), jnp.float32),
        scratch_types=[
            pltpu.VMEM((B_PER_W,), jnp.int32),
            [pltpu.VMEM((CHUNK, D_MODEL), jnp.float32) for _ in range(NBUF)],
            [pltpu.SemaphoreType.DMA for _ in range(NBUF)],
            [pltpu.SemaphoreType.DMA for _ in range(NBUF)],
        ],
        compiler_params=pltpu.CompilerParams(use_tc_tiling_on_sc=False),
    )(wview, idx2)
    return out.reshape(BATCH, HIST_PAD, D_PAD)[:, :HIST, :D_MODEL]


# speedup vs baseline: 1.5150x; 1.3548x over previous
import jax
import jax.numpy as jnp
from jax import lax
from jax.experimental import pallas as pl
from jax.experimental.pallas import tpu as pltpu
from jax.experimental.pallas import tpu_sc as plsc

D_MODEL = 64
BATCH = 16384
HIST = 50
HIST_PAD = 56
D_PAD = 128
B_TOTAL = BATCH * HIST
NUM_CORES = 2
NUM_SUBCORES = 16
NW = NUM_CORES * NUM_SUBCORES
B_PER_W = B_TOTAL // NW
XROW_PER_W = BATCH // NW
XROW_CHUNK = 8
CHUNK = XROW_CHUNK * HIST
N_CHUNK = XROW_PER_W // XROW_CHUNK
NBUF = 3


def _emb_body(table_hbm, idx_hbm, out_hbm, idx_v, rows_v, sem_g, sem_o):
    wid = lax.axis_index("s") * NUM_CORES + lax.axis_index("c")
    base = wid * B_PER_W
    xbase = wid * XROW_PER_W

    pltpu.sync_copy(idx_hbm.at[pl.ds(base, B_PER_W)], idx_v)

    def idx_slice(j):
        return idx_v.at[pl.ds(j * CHUNK, CHUNK)]

    def start_gather(j, b):
        pltpu.async_copy(table_hbm.at[idx_slice(j)], rows_v[b], sem_g[b])

    def wait_gather(j, b):
        pltpu.make_async_copy(
            table_hbm.at[idx_slice(j)], rows_v[b], sem_g[b]).wait()

    def out_pairs(j, b):
        for i in range(XROW_CHUNK):
            row0 = (xbase + j * XROW_CHUNK + i) * HIST_PAD
            src = rows_v[b].at[pl.ds(i * HIST, HIST), :]
            dst = out_hbm.at[pl.ds(row0, HIST), pl.ds(0, D_MODEL)]
            yield src, dst

    def start_out(j, b):
        for src, dst in out_pairs(j, b):
            pltpu.async_copy(src, dst, sem_o[b])

    def wait_out(j, b):
        for src, dst in out_pairs(j, b):
            pltpu.make_async_copy(src, dst, sem_o[b]).wait()

    for b in range(NBUF - 1):
        start_gather(b, b)

    def step(g, carry):
        for b in range(NBUF):
            j = g * NBUF + b

            @pl.when(j < N_CHUNK)
            def _():
                wait_gather(j, b)
                start_out(j, b)
                bp = (b - 1) % NBUF
                jn = j + NBUF - 1

                @pl.when(jn < N_CHUNK)
                def _():
                    @pl.when(j >= 1)
                    def _():
                        wait_out(j - 1, bp)

                    start_gather(jn, bp)

        return carry

    lax.fori_loop(0, pl.cdiv(N_CHUNK, NBUF), step, 0)

    for b in range(NBUF):
        j_last = N_CHUNK - NBUF + b
        wait_out(j_last, j_last % NBUF)


TR_BK = 8192


def _tr_body(wt_ref, out_ref):
    out_ref[:, :D_MODEL] = wt_ref[...].T
    out_ref[:, D_MODEL:] = jnp.zeros((TR_BK, D_PAD - D_MODEL), jnp.float32)


def _pad_transpose(wt):
    grid = pl.cdiv(1000000, TR_BK)
    return pl.pallas_call(
        _tr_body,
        grid=(grid,),
        in_specs=[pl.BlockSpec((D_MODEL, TR_BK), lambda i: (0, i))],
        out_specs=pl.BlockSpec((TR_BK, D_PAD), lambda i: (i, 0)),
        out_shape=jax.ShapeDtypeStruct((1000000, D_PAD), jnp.float32),
    )(wt)


def kernel(x, weight):
    idx2 = x.reshape(-1) * 2
    wview = _pad_transpose(weight.T).reshape(2 * 1000000, D_MODEL)
    mesh = plsc.VectorSubcoreMesh(core_axis_name="c", subcore_axis_name="s")
    out = pl.kernel(
        _emb_body,
        mesh=mesh,
        out_type=jax.ShapeDtypeStruct((BATCH * HIST_PAD, D_PAD), jnp.float32),
        scratch_types=[
            pltpu.VMEM((B_PER_W,), jnp.int32),
            [pltpu.VMEM((CHUNK, D_MODEL), jnp.float32) for _ in range(NBUF)],
            [pltpu.SemaphoreType.DMA for _ in range(NBUF)],
            [pltpu.SemaphoreType.DMA for _ in range(NBUF)],
        ],
        compiler_params=pltpu.CompilerParams(use_tc_tiling_on_sc=False),
    )(wview, idx2)
    return out.reshape(BATCH, HIST_PAD, D_PAD)[:, :HIST, :D_MODEL]
